# baseline (speedup 1.0000x reference)
import functools

import jax
import jax.numpy as jnp
from jax import lax
from jax.experimental import pallas as pl
from jax.experimental.pallas import tpu as pltpu
from jax.experimental.pallas import tpu_sc as plsc

N_NODES = 10000
N_HEDGES = 5000
NNZ = 320000
D = 128
F = 64
DW = 80

NC = 2
NS = 16
NW = NC * NS

CHUNK = 128
NBUF = 4
NNZ_PAD = 327680
N_CHUNKS = NNZ_PAD // CHUNK
CPT0 = 124
CPT1 = 36
CPT_MAX = 124
N0_TOTAL = CPT0 * NS
IDX_ROWS = N_CHUNKS + CPT_MAX

T_PAD = 10240
RPT = T_PAD // NS

_MESH = plsc.VectorSubcoreMesh(
    core_axis_name="c", subcore_axis_name="s", num_cores=NC, num_subcores=NS
)


@functools.partial(
    pl.kernel,
    out_type=jax.ShapeDtypeStruct((NC * T_PAD, DW), jnp.float32),
    mesh=_MESH,
    compiler_params=pltpu.CompilerParams(use_tc_tiling_on_sc=False),
    scratch_types=[
        pltpu.VMEM((CPT_MAX, CHUNK), jnp.int32),
        pltpu.VMEM((CPT_MAX, CHUNK), jnp.int32),
        [pltpu.VMEM((CHUNK, DW), jnp.float32)] * NBUF,
        [pltpu.SemaphoreType.DMA] * NBUF,
        pltpu.VMEM_SHARED((T_PAD, DW), jnp.float32),
    ],
)
def _sc_agg(src_hbm, gidx_hbm, sidx_hbm, zeros_hbm, out_hbm,
            gidx_v, sidx_v, rows, gsems, acc):
    c = lax.axis_index("c")
    s = lax.axis_index("s")

    pltpu.sync_copy(zeros_hbm, acc.at[pl.ds(s * RPT, RPT)])
    plsc.subcore_barrier()

    base = jnp.where(c == 0, s * CPT0, N0_TOTAL + s * CPT1)
    n_groups = jnp.where(c == 0, CPT0 // NBUF, CPT1 // NBUF)
    pltpu.sync_copy(gidx_hbm.at[pl.ds(base, CPT_MAX)], gidx_v)
    pltpu.sync_copy(sidx_hbm.at[pl.ds(base, CPT_MAX)], sidx_v)

    def group_body(g, carry):
        @pl.when(g < n_groups)
        def _():
            j0 = g * NBUF
            descs = [
                pltpu.async_copy(src_hbm.at[gidx_v.at[j0 + b]], rows[b],
                                 gsems[b])
                for b in range(NBUF)
            ]
            for b in range(NBUF):
                descs[b].wait()
                pltpu.sync_copy(rows[b], acc.at[sidx_v.at[j0 + b]], add=True)

        return carry

    lax.fori_loop(0, CPT_MAX // NBUF, group_body, 0)
    plsc.subcore_barrier()

    pltpu.sync_copy(acc.at[pl.ds(s * RPT, RPT)],
                    out_hbm.at[pl.ds(c * T_PAD + s * RPT, RPT)])


BM = 1024


def _mm_body(x_ref, w_ref, lo_ref, hi_ref):
    xp = lax.dot_general(x_ref[...], w_ref[...],
                         (((1,), (1,)), ((), ())),
                         preferred_element_type=jnp.float32)
    z = jnp.zeros((xp.shape[0], DW - F), jnp.float32)
    col = lax.broadcasted_iota(jnp.int32, (xp.shape[0], DW), 1)
    lo_ref[...] = jnp.where(col == F, 1.0,
                            jnp.concatenate([xp[:, :F], z], axis=1))
    hi_ref[...] = jnp.concatenate([xp[:, F:], z], axis=1)


def _edge_combine_body(alo_ref, blo_ref, ahi_ref, bhi_ref, wd_ref,
                       lo_ref, hi_ref):
    i = pl.program_id(0)
    slo = alo_ref[...] + blo_ref[...]
    shi = ahi_ref[...] + bhi_ref[...]
    cnt = slo[:, F:F + 1]
    scale = wd_ref[...] / jnp.maximum(cnt, 1.0)
    nrow = slo.shape[0]
    col = lax.broadcasted_iota(jnp.int32, (nrow, DW), 1)
    row = i * nrow + lax.broadcasted_iota(jnp.int32, (nrow, DW), 0)
    z = jnp.zeros((nrow, DW - F), jnp.float32)
    lo = jnp.concatenate([slo[:, :F] * scale, z], axis=1)
    lo_ref[...] = jnp.where((col == F) & (row < N_HEDGES), 1.0, lo)
    hi_ref[...] = jnp.concatenate([shi[:, :F] * scale, z], axis=1)


def _node_combine_body(alo_ref, blo_ref, ahi_ref, bhi_ref, o_ref):
    slo = alo_ref[...] + blo_ref[...]
    shi = ahi_ref[...] + bhi_ref[...]
    inv = 1.0 / jnp.maximum(slo[:, F:F + 1], 1.0)
    o_ref[...] = jnp.concatenate([slo[:, :F], shi[:, :F]], axis=1) * inv


def kernel(X, W, Wdiag, node_idx, edge_idx):
    ni = node_idx.astype(jnp.int32)
    ei = edge_idx.astype(jnp.int32)
    npad = NNZ_PAD - NNZ
    pad_n = jnp.full((npad,), N_NODES, jnp.int32)
    pad_e = jnp.full((npad,), N_HEDGES, jnp.int32)

    tail_n = jnp.full((CPT_MAX * CHUNK,), N_NODES, jnp.int32)
    tail_e = jnp.full((CPT_MAX * CHUNK,), N_HEDGES, jnp.int32)
    g1 = jnp.concatenate([ni, pad_n, tail_n]).reshape(IDX_ROWS, CHUNK)
    s1 = jnp.concatenate([ei, pad_e, tail_e]).reshape(IDX_ROWS, CHUNK)
    g2 = jnp.concatenate([ei, pad_e, tail_e]).reshape(IDX_ROWS, CHUNK)
    s2 = jnp.concatenate([ni, pad_n, tail_n]).reshape(IDX_ROWS, CHUNK)

    x_pad = jnp.zeros((T_PAD, D), jnp.float32).at[:N_NODES].set(X)
    wd_pad = jnp.zeros((T_PAD, 1), jnp.float32).at[:N_HEDGES, 0].set(Wdiag)
    zeros_blk = jnp.zeros((RPT, DW), jnp.float32)

    xp_lo, xp_hi = pl.pallas_call(
        _mm_body,
        grid=(T_PAD // BM,),
        in_specs=[
            pl.BlockSpec((BM, D), lambda i: (i, 0)),
            pl.BlockSpec((D, D), lambda i: (0, 0)),
        ],
        out_specs=[
            pl.BlockSpec((BM, DW), lambda i: (i, 0)),
            pl.BlockSpec((BM, DW), lambda i: (i, 0)),
        ],
        out_shape=[
            jax.ShapeDtypeStruct((T_PAD, DW), jnp.float32),
            jax.ShapeDtypeStruct((T_PAD, DW), jnp.float32),
        ],
    )(x_pad, W)

    pe_lo = _sc_agg(xp_lo, g1, s1, zeros_blk).reshape(NC, T_PAD, DW)
    pe_hi = _sc_agg(xp_hi, g1, s1, zeros_blk).reshape(NC, T_PAD, DW)

    eb = 1280
    xe_lo, xe_hi = pl.pallas_call(
        _edge_combine_body,
        grid=(T_PAD // eb,),
        in_specs=[pl.BlockSpec((eb, DW), lambda i: (i, 0))] * 4
        + [pl.BlockSpec((eb, 1), lambda i: (i, 0))],
        out_specs=[pl.BlockSpec((eb, DW), lambda i: (i, 0))] * 2,
        out_shape=[
            jax.ShapeDtypeStruct((T_PAD, DW), jnp.float32),
            jax.ShapeDtypeStruct((T_PAD, DW), jnp.float32),
        ],
    )(pe_lo[0], pe_lo[1], pe_hi[0], pe_hi[1], wd_pad)

    pv_lo = _sc_agg(xe_lo, g2, s2, zeros_blk).reshape(NC, T_PAD, DW)
    pv_hi = _sc_agg(xe_hi, g2, s2, zeros_blk).reshape(NC, T_PAD, DW)

    vb = 1280
    xv = pl.pallas_call(
        _node_combine_body,
        grid=(T_PAD // vb,),
        in_specs=[pl.BlockSpec((vb, DW), lambda i: (i, 0))] * 4,
        out_specs=pl.BlockSpec((vb, D), lambda i: (i, 0)),
        out_shape=jax.ShapeDtypeStruct((T_PAD, D), jnp.float32),
    )(pv_lo[0], pv_lo[1], pv_hi[0], pv_hi[1])

    return xv[:N_NODES]


# speedup vs baseline: 1.2387x; 1.2387x over previous
import functools

import jax
import jax.numpy as jnp
from jax import lax
from jax.experimental import pallas as pl
from jax.experimental.pallas import tpu as pltpu
from jax.experimental.pallas import tpu_sc as plsc

N_NODES = 10000
N_HEDGES = 5000
NNZ = 320000
D = 128

NC = 2
NS = 16
NW = NC * NS

CHUNK = 128
NNZ_PAD = 327680
N_CHUNKS = NNZ_PAD // CHUNK
CPW = N_CHUNKS // NW

N_SRC = 10240
E_PAD = 5120
V_PAD = 10240

C_ROWS = V_PAD + E_PAD
C_CHUNKS = 2 * N_CHUNKS
C_CPW = C_CHUNKS // NW
C_RPT = C_ROWS // NS
CW = 8

_MESH = plsc.VectorSubcoreMesh(
    core_axis_name="c", subcore_axis_name="s", num_cores=NC, num_subcores=NS
)


def _make_sc_aggregate(t_pad):
    rpt = t_pad // NS

    @functools.partial(
        pl.kernel,
        out_type=jax.ShapeDtypeStruct((NC * t_pad, D), jnp.float32),
        mesh=_MESH,
        compiler_params=pltpu.CompilerParams(use_tc_tiling_on_sc=False),
        scratch_types=[
            pltpu.VMEM((CPW, CHUNK), jnp.int32),
            pltpu.VMEM((CPW, CHUNK), jnp.int32),
            pltpu.VMEM((CHUNK, D), jnp.float32),
            pltpu.SemaphoreType.DMA,
            pltpu.VMEM_SHARED((t_pad, D), jnp.float32),
        ],
    )
    def agg(src_hbm, gidx_hbm, sidx_hbm, zeros_hbm, out_hbm,
            gidx_v, sidx_v, rows_v, sem, acc):
        c = lax.axis_index("c")
        s = lax.axis_index("s")
        wid = c * NS + s

        for off in range(0, rpt, 640):
            sz = min(640, rpt - off)
            pltpu.sync_copy(zeros_hbm.at[pl.ds(0, sz)],
                            acc.at[pl.ds(s * rpt + off, sz)])

        base = wid * CPW
        pltpu.sync_copy(gidx_hbm.at[pl.ds(base, CPW)], gidx_v)
        pltpu.sync_copy(sidx_hbm.at[pl.ds(base, CPW)], sidx_v)
        plsc.subcore_barrier()

        def chunk_body(j, carry):
            pltpu.async_copy(src_hbm.at[gidx_v.at[j]], rows_v, sem).wait()
            pltpu.sync_copy(rows_v, acc.at[sidx_v.at[j]], add=True)
            return carry

        lax.fori_loop(0, CPW, chunk_body, 0)
        plsc.subcore_barrier()

        pltpu.sync_copy(acc.at[pl.ds(s * rpt, rpt)],
                        out_hbm.at[pl.ds(c * t_pad + s * rpt, rpt)])

    return agg


_sc_agg_edges = _make_sc_aggregate(E_PAD)
_sc_agg_nodes = _make_sc_aggregate(V_PAD)


@functools.partial(
    pl.kernel,
    out_type=jax.ShapeDtypeStruct((NC * C_ROWS, CW), jnp.float32),
    mesh=_MESH,
    compiler_params=pltpu.CompilerParams(use_tc_tiling_on_sc=False),
    scratch_types=[
        pltpu.VMEM((C_CPW, CHUNK), jnp.int32),
        pltpu.VMEM((CHUNK, CW), jnp.float32),
        pltpu.VMEM_SHARED((C_ROWS, CW), jnp.float32),
    ],
)
def _sc_count(sidx_hbm, ones_hbm, zeros_hbm, out_hbm, sidx_v, ones_v, acc):
    c = lax.axis_index("c")
    s = lax.axis_index("s")
    wid = c * NS + s

    pltpu.sync_copy(zeros_hbm, acc.at[pl.ds(s * C_RPT, C_RPT)])
    pltpu.sync_copy(ones_hbm, ones_v)
    pltpu.sync_copy(sidx_hbm.at[pl.ds(wid * C_CPW, C_CPW)], sidx_v)
    plsc.subcore_barrier()

    def chunk_body(j, carry):
        pltpu.sync_copy(ones_v, acc.at[sidx_v.at[j]], add=True)
        return carry

    lax.fori_loop(0, C_CPW, chunk_body, 0)
    plsc.subcore_barrier()

    pltpu.sync_copy(acc.at[pl.ds(s * C_RPT, C_RPT)],
                    out_hbm.at[pl.ds(c * C_ROWS + s * C_RPT, C_RPT)])


BM = 1024


def _mm_body(x_ref, w_ref, o_ref):
    o_ref[...] = lax.dot_general(x_ref[...], w_ref[...],
                                 (((1,), (1,)), ((), ())),
                                 preferred_element_type=jnp.float32)


def _edge_combine_body(a_ref, b_ref, wd_ref, ca_ref, cb_ref, o_ref):
    cnt = ca_ref[...] + cb_ref[...]
    scale = wd_ref[...] / jnp.maximum(cnt, 1.0)
    o_ref[...] = (a_ref[...] + b_ref[...]) * scale


def _node_combine_body(a_ref, b_ref, ca_ref, cb_ref, o_ref):
    cnt = ca_ref[...] + cb_ref[...]
    o_ref[...] = (a_ref[...] + b_ref[...]) / jnp.maximum(cnt, 1.0)


def kernel(X, W, Wdiag, node_idx, edge_idx):
    ni = node_idx.astype(jnp.int32)
    ei = edge_idx.astype(jnp.int32)
    npad = NNZ_PAD - NNZ
    pad_n = jnp.full((npad,), N_NODES, jnp.int32)
    pad_e = jnp.full((npad,), N_HEDGES, jnp.int32)

    sn = jnp.concatenate([ni, pad_n])
    se = jnp.concatenate([ei, pad_e])
    g1 = jnp.concatenate([ni, pad_n]).reshape(N_CHUNKS, CHUNK)
    s1 = se.reshape(N_CHUNKS, CHUNK)
    g2 = jnp.concatenate([ei, pad_e]).reshape(N_CHUNKS, CHUNK)
    s2 = sn.reshape(N_CHUNKS, CHUNK)
    cidx = jnp.concatenate([sn, se + V_PAD]).reshape(C_CHUNKS, CHUNK)

    x_pad = jnp.zeros((N_SRC, D), jnp.float32).at[:N_NODES].set(X)
    wd_pad = jnp.zeros((E_PAD, 1), jnp.float32).at[:N_HEDGES, 0].set(Wdiag)
    zeros_blk = jnp.zeros((640, D), jnp.float32)
    zeros_cnt = jnp.zeros((C_RPT, CW), jnp.float32)
    ones_cnt = jnp.zeros((CHUNK, CW), jnp.float32).at[:, 0].set(1.0)

    pc = _sc_count(cidx, ones_cnt, zeros_cnt).reshape(NC, C_ROWS, CW)[:, :, :1]
    cv0, cv1 = pc[0, :V_PAD], pc[1, :V_PAD]
    ce0, ce1 = pc[0, V_PAD:V_PAD + E_PAD], pc[1, V_PAD:V_PAD + E_PAD]

    xp = pl.pallas_call(
        _mm_body,
        grid=(N_SRC // BM,),
        in_specs=[
            pl.BlockSpec((BM, D), lambda i: (i, 0)),
            pl.BlockSpec((D, D), lambda i: (0, 0)),
        ],
        out_specs=pl.BlockSpec((BM, D), lambda i: (i, 0)),
        out_shape=jax.ShapeDtypeStruct((N_SRC, D), jnp.float32),
    )(x_pad, W)

    pe = _sc_agg_edges(xp, g1, s1, zeros_blk).reshape(NC, E_PAD, D)

    eb = 640
    xe = pl.pallas_call(
        _edge_combine_body,
        grid=(E_PAD // eb,),
        in_specs=[
            pl.BlockSpec((eb, D), lambda i: (i, 0)),
            pl.BlockSpec((eb, D), lambda i: (i, 0)),
            pl.BlockSpec((eb, 1), lambda i: (i, 0)),
            pl.BlockSpec((eb, 1), lambda i: (i, 0)),
            pl.BlockSpec((eb, 1), lambda i: (i, 0)),
        ],
        out_specs=pl.BlockSpec((eb, D), lambda i: (i, 0)),
        out_shape=jax.ShapeDtypeStruct((E_PAD, D), jnp.float32),
    )(pe[0], pe[1], wd_pad, ce0, ce1)

    pv = _sc_agg_nodes(xe, g2, s2, zeros_blk).reshape(NC, V_PAD, D)

    vb = 1280
    xv = pl.pallas_call(
        _node_combine_body,
        grid=(V_PAD // vb,),
        in_specs=[
            pl.BlockSpec((vb, D), lambda i: (i, 0)),
            pl.BlockSpec((vb, D), lambda i: (i, 0)),
            pl.BlockSpec((vb, 1), lambda i: (i, 0)),
            pl.BlockSpec((vb, 1), lambda i: (i, 0)),
        ],
        out_specs=pl.BlockSpec((vb, D), lambda i: (i, 0)),
        out_shape=jax.ShapeDtypeStruct((V_PAD, D), jnp.float32),
    )(pv[0], pv[1], cv0, cv1)

    return xv[:N_NODES]
